# Initial kernel scaffold; baseline (speedup 1.0000x reference)
import jax
import jax.numpy as jnp
from jax.experimental import pallas as pl


def kernel(x, edge_index, edge_weight, W1, b1, W2, b2, W3, b3):
    raise NotImplementedError("write your pallas kernel here")


# speedup vs baseline: 8.4856x; 8.4856x over previous
import dataclasses
import functools

import jax
import jax.numpy as jnp
from jax import lax
from jax.experimental import pallas as pl
from jax.experimental.pallas import tpu as pltpu
from jax.experimental.pallas import tpu_sc as plsc

N = 10000
D = 128
NC = 2
NS = 16
L = 16
NW = NC * NS
CH = 128
EPW = 10368
E_PAD = EPW * NW
NCHUNK = EPW // CH
N_PAD = 10240
RPS = N // NS
NP_PS = N_PAD // NS

_mesh = plsc.VectorSubcoreMesh(core_axis_name="c", subcore_axis_name="s")

_sc_params = pltpu.CompilerParams()
if "needs_layout_passes" in pltpu.CompilerParams.__dataclass_fields__:
    _sc_params = dataclasses.replace(_sc_params, needs_layout_passes=False)


@functools.partial(
    pl.kernel, mesh=_mesh,
    out_type=jax.ShapeDtypeStruct((NC, N_PAD, D), jnp.float32),
    compiler_params=_sc_params,
    scratch_types=[
        pltpu.VMEM_SHARED((N_PAD, D), jnp.float32),
        pltpu.VMEM((CH, D), jnp.float32),
        pltpu.VMEM((1, CH), jnp.int32),
        pltpu.VMEM((1, CH), jnp.float32),
    ],
)
def _deg_kernel(dst_hbm, ew_hbm, out_hbm, acc, stg, dstb, ewb):
    c = lax.axis_index("c")
    s = lax.axis_index("s")

    @pl.loop(0, CH)
    def _(i):
        for q in range(D // L):
            stg[i, pl.ds(q * L, L)] = jnp.zeros((L,), jnp.float32)

    @pl.loop(0, NP_PS // CH)
    def _(t):
        pltpu.sync_copy(stg, acc.at[pl.ds(s * NP_PS + t * CH, CH), :])

    plsc.subcore_barrier()

    base = (c * NS + s) * EPW
    rows = lax.iota(jnp.int32, L)
    zcol = jnp.zeros((L,), jnp.int32)

    @pl.loop(0, NCHUNK)
    def _(k):
        off = base + k * CH
        pltpu.sync_copy(dst_hbm.at[pl.ds(off, CH)], dstb.at[0])
        pltpu.sync_copy(ew_hbm.at[pl.ds(off, CH)], ewb.at[0])
        for g in range(CH // L):
            ewv = ewb[0, pl.ds(g * L, L)]
            plsc.store_scatter(stg, [rows + g * L, zcol], ewv)
        pltpu.sync_copy(stg, acc.at[dstb.at[0]], add=True)

    plsc.subcore_barrier()

    @pl.loop(0, NP_PS // CH)
    def _(t):
        r0 = s * NP_PS + t * CH
        pltpu.sync_copy(acc.at[pl.ds(r0, CH), :],
                        out_hbm.at[c, pl.ds(r0, CH), :])


@functools.partial(
    pl.kernel, mesh=_mesh,
    out_type=jax.ShapeDtypeStruct((NC, N, D), jnp.float32),
    compiler_params=_sc_params,
    scratch_types=[
        pltpu.VMEM_SHARED((N, D), jnp.float32),
        pltpu.VMEM((CH, D), jnp.float32),
        pltpu.VMEM((1, CH), jnp.int32),
        pltpu.VMEM((1, CH), jnp.int32),
        pltpu.VMEM((1, CH), jnp.float32),
        pltpu.VMEM((CH,), jnp.float32),
        pltpu.VMEM((N_PAD,), jnp.float32),
    ],
)
def _agg_kernel(g_hbm, src_hbm, dst_hbm, ew_hbm, dinv_hbm, out_hbm,
                acc, rowb, srcb, dstb, ewb, normb, dinvb):
    c = lax.axis_index("c")
    s = lax.axis_index("s")

    @pl.loop(0, CH)
    def _(i):
        for q in range(D // L):
            rowb[i, pl.ds(q * L, L)] = jnp.zeros((L,), jnp.float32)

    @pl.loop(0, 8)
    def _(t):
        r0 = s * 640 + t * 80

        @pl.when(r0 < N)
        def _():
            pltpu.sync_copy(rowb.at[pl.ds(0, 80), :],
                            acc.at[pl.ds(r0, 80), :])

    pltpu.sync_copy(dinv_hbm, dinvb)
    plsc.subcore_barrier()

    base = (c * NS + s) * EPW

    @pl.loop(0, NCHUNK)
    def _(k):
        off = base + k * CH
        pltpu.sync_copy(src_hbm.at[pl.ds(off, CH)], srcb.at[0])
        pltpu.sync_copy(dst_hbm.at[pl.ds(off, CH)], dstb.at[0])
        pltpu.sync_copy(ew_hbm.at[pl.ds(off, CH)], ewb.at[0])
        pltpu.sync_copy(g_hbm.at[srcb.at[0]], rowb)

        for g in range(CH // L):
            srcv = srcb[0, pl.ds(g * L, L)]
            dstv = dstb[0, pl.ds(g * L, L)]
            ewv = ewb[0, pl.ds(g * L, L)]
            nv = (plsc.load_gather(dinvb, [srcv]) * ewv
                  * plsc.load_gather(dinvb, [dstv]))
            normb[pl.ds(g * L, L)] = nv

        @pl.loop(0, CH // L)
        def _(gi):
            nvec = normb[pl.ds(gi * L, L)]
            for lane in range(L):
                nb = lax.broadcast_in_dim(nvec[lane], (L,), ())
                for q in range(D // L):
                    e = gi * L + lane
                    rowb[e, pl.ds(q * L, L)] = rowb[e, pl.ds(q * L, L)] * nb

        pltpu.sync_copy(rowb, acc.at[dstb.at[0]], add=True)

    plsc.subcore_barrier()

    @pl.loop(0, 8)
    def _(t):
        r0 = s * 640 + t * 80

        @pl.when(r0 < N)
        def _():
            pltpu.sync_copy(acc.at[pl.ds(r0, 80), :],
                            out_hbm.at[c, pl.ds(r0, 80), :])


def _mm1_dinv(x, W1, degp):
    def body(x_ref, w_ref, d_ref, g_ref, dinv_ref):
        i = pl.program_id(0)
        g_ref[...] = jnp.dot(x_ref[...], w_ref[...],
                             preferred_element_type=jnp.float32)

        @pl.when(i == 0)
        def _():
            dsum = d_ref[0] + d_ref[1]
            dinv_ref[...] = jnp.where(
                dsum > 0, lax.rsqrt(jnp.maximum(dsum, 1e-12)), 0.0)

    return pl.pallas_call(
        body,
        grid=(10,),
        in_specs=[pl.BlockSpec((1000, D), lambda i: (i, 0)),
                  pl.BlockSpec((D, D), lambda i: (0, 0)),
                  pl.BlockSpec((2, 80, 128), lambda i: (0, 0, 0))],
        out_specs=[pl.BlockSpec((1000, D), lambda i: (i, 0)),
                   pl.BlockSpec((80, 128), lambda i: (0, 0))],
        out_shape=[jax.ShapeDtypeStruct((N, D), jnp.float32),
                   jax.ShapeDtypeStruct((80, 128), jnp.float32)],
    )(x, W1, degp)


def _mm_fused(p, b, W):
    def body(p0_ref, p1_ref, b_ref, w_ref, o_ref):
        h = jnp.maximum(p0_ref[0] + p1_ref[0] + b_ref[...], 0.0)
        o_ref[...] = jnp.dot(h, w_ref[...],
                             preferred_element_type=jnp.float32)

    return pl.pallas_call(
        body,
        grid=(10,),
        in_specs=[pl.BlockSpec((1, 1000, D), lambda i: (0, i, 0)),
                  pl.BlockSpec((1, 1000, D), lambda i: (1, i, 0)),
                  pl.BlockSpec((1, D), lambda i: (0, 0)),
                  pl.BlockSpec((D, D), lambda i: (0, 0))],
        out_specs=pl.BlockSpec((1000, D), lambda i: (i, 0)),
        out_shape=jax.ShapeDtypeStruct((N, D), jnp.float32),
    )(p, p, b, W)


def _final(p, b):
    def body(p0_ref, p1_ref, b_ref, o_ref):
        o_ref[...] = p0_ref[0] + p1_ref[0] + b_ref[...]

    return pl.pallas_call(
        body,
        grid=(10,),
        in_specs=[pl.BlockSpec((1, 1000, D), lambda i: (0, i, 0)),
                  pl.BlockSpec((1, 1000, D), lambda i: (1, i, 0)),
                  pl.BlockSpec((1, D), lambda i: (0, 0))],
        out_specs=pl.BlockSpec((1000, D), lambda i: (i, 0)),
        out_shape=jax.ShapeDtypeStruct((N, D), jnp.float32),
    )(p, p, b)


def kernel(x, edge_index, edge_weight, W1, b1, W2, b2, W3, b3):
    loop_idx = jnp.arange(N, dtype=edge_index.dtype)
    src = jnp.concatenate([edge_index[0], loop_idx])
    dst = jnp.concatenate([edge_index[1], loop_idx])
    ew = jnp.concatenate([edge_weight, jnp.ones((N,), edge_weight.dtype)])
    pad = E_PAD - src.shape[0]
    src = jnp.pad(src, (0, pad))
    dst = jnp.pad(dst, (0, pad))
    ew = jnp.pad(ew, (0, pad))

    degp = _deg_kernel(dst, ew)[:, :, 0].reshape(NC, 80, 128)
    g1, dinv2d = _mm1_dinv(x, W1, degp)
    dinv = dinv2d.reshape(N_PAD)
    p = _agg_kernel(g1, src, dst, ew, dinv)
    g2 = _mm_fused(p, b1.reshape(1, D), W2)
    p = _agg_kernel(g2, src, dst, ew, dinv)
    g3 = _mm_fused(p, b2.reshape(1, D), W3)
    p = _agg_kernel(g3, src, dst, ew, dinv)
    return _final(p, b3.reshape(1, D))
